# baseline (speedup 1.0000x reference)
import dataclasses
import functools

import jax
import jax.numpy as jnp
from jax import lax
from jax.experimental import pallas as pl
from jax.experimental.pallas import tpu as pltpu
from jax.experimental.pallas import tpu_sc as plsc

_N = 100000
_E = 6400000
_C = 4
_CP = 8
_CHUNK = 1024
_SUB = _CHUNK // 2
_KR = _SUB // 128
_NCHUNKS = _E // _CHUNK
_NWORKERS = 32
_MAXT = -(-_NCHUNKS // _NWORKERS)
_S0 = 6256
_SLAST = _N - 15 * _S0
_BLK = 512


def _sc_accumulate(charges, nbr, dist):
    mesh = plsc.VectorSubcoreMesh(core_axis_name="c", subcore_axis_name="s")
    cp = pltpu.CompilerParams()
    if "needs_layout_passes" in pltpu.CompilerParams.__dataclass_fields__:
        cp = dataclasses.replace(cp, needs_layout_passes=False)
    if "use_tc_tiling_on_sc" in pltpu.CompilerParams.__dataclass_fields__:
        cp = dataclasses.replace(cp, use_tc_tiling_on_sc=False)

    sub_set = [
        pltpu.VMEM((_KR, 128), jnp.int32),
        pltpu.VMEM((_KR, 128), jnp.int32),
        pltpu.VMEM((2 * _KR, 128), jnp.int32),
        pltpu.VMEM((_SUB,), jnp.float32),
        pltpu.VMEM((_SUB, _CP), jnp.float32),
        pltpu.VMEM((_SUB, _CP), jnp.float32),
        pltpu.SemaphoreType.DMA,
        pltpu.SemaphoreType.DMA,
        pltpu.SemaphoreType.DMA,
    ]

    @functools.partial(
        pl.kernel,
        compiler_params=cp,
        out_type=jax.ShapeDtypeStruct((2, _N, _CP), jnp.float32),
        mesh=mesh,
        scratch_types=[
            pltpu.VMEM_SHARED((_N, _CP), jnp.float32),
            pltpu.VMEM_SHARED((_N, _CP), jnp.float32),
            pltpu.VMEM((_BLK, _C), jnp.float32),
        ] + sub_set + sub_set,
    )
    def k(chg_hbm, nbr_hbm, dist_hbm, out_hbm,
          chg_sh, acc_sh, c4,
          ii0, jj0, pr0, ww0, ba0, bb0, si0, sg0, ss0,
          ii1, jj1, pr1, ww1, ba1, bb1, si1, sg1, ss1):
        c = lax.axis_index("c")
        s = lax.axis_index("s")
        wid = c * 16 + s

        iota = lax.iota(jnp.int32, 16)
        four = jnp.full((16,), _C, jnp.int32)
        col = lax.rem(iota, four)
        rpat = lax.div(iota, four)
        iota2 = iota * 2
        zero16 = jnp.zeros((16,), jnp.float32)
        row8 = lax.div(iota, jnp.full((16,), _CP, jnp.int32))
        col8 = lax.rem(iota, jnp.full((16,), _CP, jnp.int32))

        @pl.loop(0, _BLK * _CP // 16)
        def _z(g):
            plsc.store_scatter(bb0, [row8 + g * 2, col8], zero16)

        def stage_block(off, nrows):
            pltpu.sync_copy(chg_hbm.at[pl.ds(off, nrows)],
                            c4.at[pl.ds(0, nrows)])

            @pl.loop(0, nrows * _C // 16)
            def _p(g):
                r = rpat + g * 4
                v = plsc.load_gather(c4, [r, col])
                plsc.store_scatter(ba0, [r, col], v)

            pltpu.sync_copy(ba0.at[pl.ds(0, nrows)],
                            chg_sh.at[pl.ds(off, nrows)])
            pltpu.sync_copy(bb0.at[pl.ds(0, nrows)],
                            acc_sh.at[pl.ds(off, nrows)])

        row0 = pl.multiple_of(s * _S0, 8)

        @pl.when(s < 15)
        def _():
            @pl.loop(0, _S0 // _BLK)
            def _b(b):
                stage_block(row0 + b * _BLK, _BLK)
            stage_block(row0 + (_S0 // _BLK) * _BLK, _S0 % _BLK)

        @pl.when(s == 15)
        def _():
            @pl.loop(0, _SLAST // _BLK)
            def _b(b):
                stage_block(15 * _S0 + b * _BLK, _BLK)
            stage_block(15 * _S0 + (_SLAST // _BLK) * _BLK,
                        _SLAST % _BLK)

        plsc.subcore_barrier()

        def issue_inputs(q, sub, pr, ww, sem):
            rb = q * (4 * _KR) + sub * (2 * _KR)
            eb = q * _CHUNK + sub * _SUB
            return (pltpu.async_copy(nbr_hbm.at[pl.ds(rb, 2 * _KR)], pr, sem),
                    pltpu.async_copy(dist_hbm.at[pl.ds(eb, _SUB)], ww, sem))

        def deinterleave(pr, ii, jj):
            @pl.loop(0, _SUB // 16)
            def _d(g):
                srow = jnp.full((16,), g // 4, jnp.int32)
                scol = iota2 + (g % 4) * 32
                iv = plsc.load_gather(pr, [srow, scol])
                jv = plsc.load_gather(pr, [srow, scol + 1])
                drow = jnp.full((16,), g // 8, jnp.int32)
                dcol = iota + (g % 8) * 16
                plsc.store_scatter(ii, [drow, dcol], iv)
                plsc.store_scatter(jj, [drow, dcol], jv)

        def issue_gathers(ii, jj, ba, bb, sem):
            hs = []
            for k2 in range(_KR):
                sl = pl.ds(k2 * 128, 128)
                hs.append(pltpu.async_copy(chg_sh.at[jj.at[k2]], ba.at[sl], sem))
                hs.append(pltpu.async_copy(chg_sh.at[ii.at[k2]], bb.at[sl], sem))
            return hs

        def compute_w(ww):
            @pl.loop(0, _SUB // 16)
            def _w(u):
                sl = pl.ds(u * 16, 16)
                ww[sl] = 0.5 / ww[sl]

        def scale(ww, ba, bb):
            @plsc.parallel_loop(0, _SUB * _C // 16, unroll=4)
            def _scale(g):
                row = rpat + g * 4
                wb = plsc.load_gather(ww, [row])
                a = plsc.load_gather(ba, [row, col])
                plsc.store_scatter(ba, [row, col], a * wb)
                b = plsc.load_gather(bb, [row, col])
                plsc.store_scatter(bb, [row, col], b * wb)

        def issue_scatters(ii, jj, ba, bb, sem):
            hs = []
            for k2 in range(_KR):
                sl = pl.ds(k2 * 128, 128)
                hs.append(pltpu.async_copy(ba.at[sl], acc_sh.at[ii.at[k2]],
                                           sem, add=True))
                hs.append(pltpu.async_copy(bb.at[sl], acc_sh.at[jj.at[k2]],
                                           sem, add=True))
            return hs

        @pl.loop(0, _MAXT)
        def _chunks(t):
            q = wid + t * _NWORKERS

            @pl.when(q < _NCHUNKS)
            def _():
                h_i0 = issue_inputs(q, 0, pr0, ww0, si0)
                h_i1 = issue_inputs(q, 1, pr1, ww1, si1)
                for h in h_i0:
                    h.wait()
                deinterleave(pr0, ii0, jj0)
                h_g0 = issue_gathers(ii0, jj0, ba0, bb0, sg0)
                compute_w(ww0)
                for h in h_i1:
                    h.wait()
                deinterleave(pr1, ii1, jj1)
                h_g1 = issue_gathers(ii1, jj1, ba1, bb1, sg1)
                compute_w(ww1)
                for h in h_g0:
                    h.wait()
                scale(ww0, ba0, bb0)
                h_s0 = issue_scatters(ii0, jj0, ba0, bb0, ss0)
                for h in h_g1:
                    h.wait()
                scale(ww1, ba1, bb1)
                h_s1 = issue_scatters(ii1, jj1, ba1, bb1, ss1)
                for h in h_s0 + h_s1:
                    h.wait()

        plsc.subcore_barrier()

        @pl.when(s < 15)
        def _():
            pltpu.sync_copy(acc_sh.at[pl.ds(row0, _S0)],
                            out_hbm.at[c].at[pl.ds(row0, _S0)])

        @pl.when(s == 15)
        def _():
            pltpu.sync_copy(acc_sh.at[pl.ds(15 * _S0, _SLAST)],
                            out_hbm.at[c].at[pl.ds(15 * _S0, _SLAST)])

    return k(charges, nbr, dist)


def _tc_combine(parts):
    def body(x_ref, o_ref):
        o_ref[...] = x_ref[0] + x_ref[1]

    return pl.pallas_call(
        body,
        out_shape=jax.ShapeDtypeStruct(parts.shape[1:], parts.dtype),
    )(parts)


def kernel(charges, cell, positions, neighbor_indices, neighbor_distances):
    del cell, positions
    nbr = neighbor_indices.reshape(_E * 2 // 128, 128)
    parts = _sc_accumulate(charges, nbr, neighbor_distances)
    out = _tc_combine(parts.reshape(2, _N * _CP // 128, 128))
    return out.reshape(_N, _CP)[:, :_C]


# speedup vs baseline: 11.9392x; 11.9392x over previous
import dataclasses
import functools

import jax
import jax.numpy as jnp
from jax import lax
from jax.experimental import pallas as pl
from jax.experimental.pallas import tpu as pltpu
from jax.experimental.pallas import tpu_sc as plsc

_N = 100000
_E = 6400000
_C = 4
_CP = 8
_CHUNK = 1024
_SUB = _CHUNK // 2
_KR = _SUB // 128
_NCHUNKS = _E // _CHUNK
_NWORKERS = 32
_MAXT = -(-_NCHUNKS // _NWORKERS)
_S0 = 6256
_SLAST = _N - 15 * _S0
_BLK = 512


def _sc_accumulate(charges, ai, aj, dist):
    mesh = plsc.VectorSubcoreMesh(core_axis_name="c", subcore_axis_name="s")
    cp = pltpu.CompilerParams()
    if "needs_layout_passes" in pltpu.CompilerParams.__dataclass_fields__:
        cp = dataclasses.replace(cp, needs_layout_passes=False)
    if "use_tc_tiling_on_sc" in pltpu.CompilerParams.__dataclass_fields__:
        cp = dataclasses.replace(cp, use_tc_tiling_on_sc=False)

    sub_set = [
        pltpu.VMEM((_KR, 128), jnp.int32),
        pltpu.VMEM((_KR, 128), jnp.int32),
        pltpu.VMEM((_SUB,), jnp.float32),
        pltpu.VMEM((_SUB, _CP), jnp.float32),
        pltpu.VMEM((_SUB, _CP), jnp.float32),
        pltpu.SemaphoreType.DMA,
        pltpu.SemaphoreType.DMA,
        pltpu.SemaphoreType.DMA,
    ]

    @functools.partial(
        pl.kernel,
        compiler_params=cp,
        out_type=jax.ShapeDtypeStruct((2, _N, _CP), jnp.float32),
        mesh=mesh,
        scratch_types=[
            pltpu.VMEM_SHARED((_N, _CP), jnp.float32),
            pltpu.VMEM_SHARED((_N, _CP), jnp.float32),
            pltpu.VMEM((_BLK, _C), jnp.float32),
        ] + sub_set + sub_set,
    )
    def k(chg_hbm, ai_hbm, aj_hbm, dist_hbm, out_hbm,
          chg_sh, acc_sh, c4,
          ii0, jj0, ww0, ba0, bb0, si0, sg0, ss0,
          ii1, jj1, ww1, ba1, bb1, si1, sg1, ss1):
        c = lax.axis_index("c")
        s = lax.axis_index("s")
        wid = c * 16 + s

        iota = lax.iota(jnp.int32, 16)
        four = jnp.full((16,), _C, jnp.int32)
        col = lax.rem(iota, four)
        rpat = lax.div(iota, four)
        zero16 = jnp.zeros((16,), jnp.float32)
        row8 = lax.div(iota, jnp.full((16,), _CP, jnp.int32))
        col8 = lax.rem(iota, jnp.full((16,), _CP, jnp.int32))

        @pl.loop(0, _BLK * _CP // 16)
        def _z(g):
            plsc.store_scatter(bb0, [row8 + g * 2, col8], zero16)

        def stage_block(off, nrows):
            pltpu.sync_copy(chg_hbm.at[pl.ds(off, nrows)],
                            c4.at[pl.ds(0, nrows)])

            @pl.loop(0, nrows * _C // 16)
            def _p(g):
                r = rpat + g * 4
                v = plsc.load_gather(c4, [r, col])
                plsc.store_scatter(ba0, [r, col], v)

            pltpu.sync_copy(ba0.at[pl.ds(0, nrows)],
                            chg_sh.at[pl.ds(off, nrows)])
            pltpu.sync_copy(bb0.at[pl.ds(0, nrows)],
                            acc_sh.at[pl.ds(off, nrows)])

        row0 = pl.multiple_of(s * _S0, 8)

        @pl.when(s < 15)
        def _():
            @pl.loop(0, _S0 // _BLK)
            def _b(b):
                stage_block(row0 + b * _BLK, _BLK)
            stage_block(row0 + (_S0 // _BLK) * _BLK, _S0 % _BLK)

        @pl.when(s == 15)
        def _():
            @pl.loop(0, _SLAST // _BLK)
            def _b(b):
                stage_block(15 * _S0 + b * _BLK, _BLK)
            stage_block(15 * _S0 + (_SLAST // _BLK) * _BLK,
                        _SLAST % _BLK)

        plsc.subcore_barrier()

        def issue_inputs(q, sub, ii, jj, ww, sem):
            rb = q * (2 * _KR) + sub * _KR
            eb = q * _CHUNK + sub * _SUB
            return (pltpu.async_copy(ai_hbm.at[pl.ds(rb, _KR)], ii, sem),
                    pltpu.async_copy(aj_hbm.at[pl.ds(rb, _KR)], jj, sem),
                    pltpu.async_copy(dist_hbm.at[pl.ds(eb, _SUB)], ww, sem))

        def issue_gathers(ii, jj, ba, bb, sem):
            hs = []
            for k2 in range(_KR):
                sl = pl.ds(k2 * 128, 128)
                hs.append(pltpu.async_copy(chg_sh.at[jj.at[k2]], ba.at[sl], sem))
                hs.append(pltpu.async_copy(chg_sh.at[ii.at[k2]], bb.at[sl], sem))
            return hs

        def compute_w(ww):
            @pl.loop(0, _SUB // 16)
            def _w(u):
                sl = pl.ds(u * 16, 16)
                ww[sl] = 0.5 / ww[sl]

        def scale(ww, ba, bb):
            @plsc.parallel_loop(0, _SUB * _C // 16, unroll=4)
            def _scale(g):
                row = rpat + g * 4
                wb = plsc.load_gather(ww, [row])
                a = plsc.load_gather(ba, [row, col])
                plsc.store_scatter(ba, [row, col], a * wb)
                b = plsc.load_gather(bb, [row, col])
                plsc.store_scatter(bb, [row, col], b * wb)

        def issue_scatters(ii, jj, ba, bb, sem):
            hs = []
            for k2 in range(_KR):
                sl = pl.ds(k2 * 128, 128)
                hs.append(pltpu.async_copy(ba.at[sl], acc_sh.at[ii.at[k2]],
                                           sem, add=True))
                hs.append(pltpu.async_copy(bb.at[sl], acc_sh.at[jj.at[k2]],
                                           sem, add=True))
            return hs

        @pl.loop(0, _MAXT)
        def _chunks(t):
            q = wid + t * _NWORKERS

            @pl.when(q < _NCHUNKS)
            def _():
                h_i0 = issue_inputs(q, 0, ii0, jj0, ww0, si0)
                h_i1 = issue_inputs(q, 1, ii1, jj1, ww1, si1)
                for h in h_i0:
                    h.wait()
                h_g0 = issue_gathers(ii0, jj0, ba0, bb0, sg0)
                compute_w(ww0)
                for h in h_i1:
                    h.wait()
                h_g1 = issue_gathers(ii1, jj1, ba1, bb1, sg1)
                compute_w(ww1)
                for h in h_g0:
                    h.wait()
                scale(ww0, ba0, bb0)
                h_s0 = issue_scatters(ii0, jj0, ba0, bb0, ss0)
                for h in h_g1:
                    h.wait()
                scale(ww1, ba1, bb1)
                h_s1 = issue_scatters(ii1, jj1, ba1, bb1, ss1)
                for h in h_s0 + h_s1:
                    h.wait()

        plsc.subcore_barrier()

        @pl.when(s < 15)
        def _():
            pltpu.sync_copy(acc_sh.at[pl.ds(row0, _S0)],
                            out_hbm.at[c].at[pl.ds(row0, _S0)])

        @pl.when(s == 15)
        def _():
            pltpu.sync_copy(acc_sh.at[pl.ds(15 * _S0, _SLAST)],
                            out_hbm.at[c].at[pl.ds(15 * _S0, _SLAST)])

    return k(charges, ai, aj, dist)


def _tc_combine(parts):
    def body(x_ref, o_ref):
        o_ref[...] = x_ref[0] + x_ref[1]

    return pl.pallas_call(
        body,
        out_shape=jax.ShapeDtypeStruct(parts.shape[1:], parts.dtype),
    )(parts)


def kernel(charges, cell, positions, neighbor_indices, neighbor_distances):
    del cell, positions
    ai = neighbor_indices[:, 0].reshape(_E // 128, 128)
    aj = neighbor_indices[:, 1].reshape(_E // 128, 128)
    parts = _sc_accumulate(charges, ai, aj, neighbor_distances)
    out = _tc_combine(parts.reshape(2, _N * _CP // 128, 128))
    return out.reshape(_N, _CP)[:, :_C]
